# baseline (speedup 1.0000x reference)
import functools

import jax
import jax.numpy as jnp
from jax import lax
from jax.experimental import pallas as pl
from jax.experimental.pallas import tpu as pltpu
from jax.experimental.pallas import tpu_sc as plsc

B, K, H, W = 8, 100, 128, 128
HW = H * W
C = 7
KP = 128
RPB = 4
NJ = K // RPB
KW = KP // 4


def _argmax_body(x_ref, pb_ref, vm_ref):
    b = pl.program_id(0)
    x = x_ref[0]
    vmax = jnp.max(x, axis=1, keepdims=True)
    iota = lax.broadcasted_iota(jnp.int32, (RPB, HW), 1)
    amin = jnp.min(jnp.where(x == vmax, iota, jnp.int32(2**30)),
                   axis=1, keepdims=True)
    pb_ref[...] = (b * (8 * HW) + amin).reshape(1, 1, RPB, 1)
    vm_ref[...] = vmax.reshape(1, 1, RPB, 1)


def _argmax_call(hm):
    return pl.pallas_call(
        _argmax_body,
        grid=(B, NJ),
        in_specs=[pl.BlockSpec((1, RPB, HW), lambda b, j: (b * NJ + j, 0, 0))],
        out_specs=[pl.BlockSpec((1, 1, RPB, 1), lambda b, j: (b, j, 0, 0))] * 2,
        out_shape=[jax.ShapeDtypeStruct((B, NJ, RPB, 1), jnp.int32),
                   jax.ShapeDtypeStruct((B, NJ, RPB, 1), jnp.float32)],
    )(hm)


def _sc_loss_body(pred_hbm, gtp_hbm, pb_hbm, vm_hbm, out_hbm, parts_hbm,
                  pb_v, vm_v, gt_v, val_v, part_v, fold_v, red_v, out_v, sem):
    cid = lax.axis_index("c")
    sid = lax.axis_index("s")
    w = sid * 2 + cid
    base = (w // 4) * KP + (w % 4) * KW

    pltpu.sync_copy(pb_hbm.at[pl.ds(base, KW)], pb_v)
    pltpu.sync_copy(vm_hbm.at[pl.ds(base, KW)], vm_v)
    for c in range(C):
        pltpu.sync_copy(gtp_hbm.at[pl.ds(c * (B * KP) + base, KW)], gt_v.at[c])

    copies = []
    for g in range(2):
        pb = pb_v[pl.ds(g * 16, 16)]
        for c in range(C):
            cp = pltpu.async_copy(pred_hbm.at[pb + c * HW], val_v.at[g * C + c], sem)
            copies.append(cp)
    for cp in copies:
        cp.wait()

    total = jnp.zeros((16,), jnp.float32)
    for g in range(2):
        acc = jnp.zeros((16,), jnp.float32)
        for c in range(C):
            d = val_v[g * C + c] - gt_v[c, pl.ds(g * 16, 16)]
            acc = acc + d * d
        vm = vm_v[pl.ds(g * 16, 16)]
        total = total + jnp.where(vm == 1.0, acc, jnp.float32(0.0))

    part_v[pl.ds(0, 16)] = total
    for sh in (8, 4, 2, 1):
        part_v[pl.ds(0, 16)] = part_v[pl.ds(0, 16)] + part_v[pl.ds(sh, 16)]
    fold_v[...] = part_v[pl.ds(0, 16)]
    pltpu.sync_copy(fold_v, parts_hbm.at[w])
    plsc.subcore_barrier()

    @pl.when(w == 0)
    def _():
        pltpu.sync_copy(parts_hbm, red_v)
        for bb in range(B):
            out_v[bb] = (red_v[4 * bb] + red_v[4 * bb + 1]
                         + red_v[4 * bb + 2] + red_v[4 * bb + 3])
        pltpu.sync_copy(out_v, out_hbm)


@functools.cache
def _sc_loss_kernel():
    mesh = plsc.VectorSubcoreMesh(core_axis_name="c", subcore_axis_name="s")
    return pl.kernel(
        _sc_loss_body,
        out_type=(jax.ShapeDtypeStruct((B, 16), jnp.float32),
                  jax.ShapeDtypeStruct((32, 16), jnp.float32)),
        mesh=mesh,
        scratch_types=[
            pltpu.VMEM((KW,), jnp.int32),
            pltpu.VMEM((KW,), jnp.float32),
            pltpu.VMEM((C, KW), jnp.float32),
            pltpu.VMEM((2 * C, 16), jnp.float32),
            pltpu.VMEM((32,), jnp.float32),
            pltpu.VMEM((16,), jnp.float32),
            pltpu.VMEM((32, 16), jnp.float32),
            pltpu.VMEM((B, 16), jnp.float32),
            pltpu.SemaphoreType.DMA,
        ],
    )


def kernel(pred, gt, heatmap):
    hm = heatmap.reshape(B * K // RPB, RPB, HW)
    pb4, vm4 = _argmax_call(hm)
    pb = jnp.pad(pb4.reshape(B, K), ((0, 0), (0, KP - K))).reshape(-1)
    vm = jnp.pad(vm4.reshape(B, K), ((0, 0), (0, KP - K))).reshape(-1)
    gtp = jnp.pad(jnp.transpose(gt[:, :, 0:C], (2, 0, 1)),
                  ((0, 0), (0, 0), (0, KP - K))).reshape(-1)
    out16, _ = _sc_loss_kernel()(pred.reshape(-1), gtp, pb, vm)
    return out16[:, 0]


# speedup vs baseline: 1.0153x; 1.0153x over previous
import functools

import jax
import jax.numpy as jnp
from jax import lax
from jax.experimental import pallas as pl
from jax.experimental.pallas import tpu as pltpu
from jax.experimental.pallas import tpu_sc as plsc

B, K, H, W = 8, 100, 128, 128
HW = H * W
C = 7
RPB = 4
NJ = K // RPB
SPT = 25


def _argmax_body(x_ref, pb_ref, vm_ref):
    b = pl.program_id(0)
    x = x_ref[0]
    vmax = jnp.max(x, axis=1, keepdims=True)
    iota = lax.broadcasted_iota(jnp.int32, (RPB, HW), 1)
    amin = jnp.min(jnp.where(x == vmax, iota, jnp.int32(2**30)),
                   axis=1, keepdims=True)
    pb_ref[...] = (b * (8 * HW) + amin).reshape(1, 1, RPB, 1)
    vm_ref[...] = vmax.reshape(1, 1, RPB, 1)


def _argmax_call(hm):
    return pl.pallas_call(
        _argmax_body,
        grid=(B, NJ),
        in_specs=[pl.BlockSpec((1, RPB, HW), lambda b, j: (b * NJ + j, 0, 0))],
        out_specs=[pl.BlockSpec((1, 1, RPB, 1), lambda b, j: (b, j, 0, 0))] * 2,
        out_shape=[jax.ShapeDtypeStruct((B, NJ, RPB, 1), jnp.int32),
                   jax.ShapeDtypeStruct((B, NJ, RPB, 1), jnp.float32)],
    )(hm)


def _sc_loss_body(pred_hbm, gt_hbm, pb_hbm, vm_hbm, out_hbm, parts_hbm,
                  pb_v, vm_v, gt_v, val_v, part_v, fold_v, red_v, out_v, sem):
    cid = lax.axis_index("c")
    sid = lax.axis_index("s")
    w = sid * 2 + cid
    base = w * SPT
    it = lax.broadcasted_iota(jnp.int32, (16,), 0)
    idx0 = base + it
    idx1 = jnp.minimum(base + 16 + it, B * K - 1)

    copies = []
    for g, idx in enumerate((idx0, idx1)):
        copies.append(pltpu.async_copy(pb_hbm.at[idx], pb_v.at[g], sem))
        copies.append(pltpu.async_copy(vm_hbm.at[idx], vm_v.at[g], sem))
        for c in range(C):
            copies.append(
                pltpu.async_copy(gt_hbm.at[idx * 8 + c], gt_v.at[g * C + c], sem))
    for cp in copies:
        cp.wait()

    copies = []
    for g in range(2):
        pb = pb_v[g]
        for c in range(C):
            copies.append(
                pltpu.async_copy(pred_hbm.at[pb + c * HW], val_v.at[g * C + c], sem))
    for cp in copies:
        cp.wait()

    total = jnp.zeros((16,), jnp.float32)
    for g in range(2):
        acc = jnp.zeros((16,), jnp.float32)
        for c in range(C):
            d = val_v[g * C + c] - gt_v[g * C + c]
            acc = acc + d * d
        live = vm_v[g] == 1.0
        if g == 1:
            live = live & (it < SPT - 16)
        total = total + jnp.where(live, acc, jnp.float32(0.0))

    part_v[pl.ds(0, 16)] = total
    for sh in (8, 4, 2, 1):
        part_v[pl.ds(0, 16)] = part_v[pl.ds(0, 16)] + part_v[pl.ds(sh, 16)]
    fold_v[...] = part_v[pl.ds(0, 16)]
    pltpu.sync_copy(fold_v, parts_hbm.at[w])
    plsc.subcore_barrier()

    @pl.when(w == 0)
    def _():
        pltpu.sync_copy(parts_hbm, red_v)
        for bb in range(B):
            out_v[bb] = (red_v[4 * bb] + red_v[4 * bb + 1]
                         + red_v[4 * bb + 2] + red_v[4 * bb + 3])
        pltpu.sync_copy(out_v, out_hbm)


@functools.cache
def _sc_loss_kernel():
    mesh = plsc.VectorSubcoreMesh(core_axis_name="c", subcore_axis_name="s")
    return pl.kernel(
        _sc_loss_body,
        out_type=(jax.ShapeDtypeStruct((B, 16), jnp.float32),
                  jax.ShapeDtypeStruct((32, 16), jnp.float32)),
        mesh=mesh,
        scratch_types=[
            pltpu.VMEM((2, 16), jnp.int32),
            pltpu.VMEM((2, 16), jnp.float32),
            pltpu.VMEM((2 * C, 16), jnp.float32),
            pltpu.VMEM((2 * C, 16), jnp.float32),
            pltpu.VMEM((32,), jnp.float32),
            pltpu.VMEM((16,), jnp.float32),
            pltpu.VMEM((32, 16), jnp.float32),
            pltpu.VMEM((B, 16), jnp.float32),
            pltpu.SemaphoreType.DMA,
        ],
    )


def kernel(pred, gt, heatmap):
    hm = heatmap.reshape(B * K // RPB, RPB, HW)
    pb4, vm4 = _argmax_call(hm)
    out16, _ = _sc_loss_kernel()(
        pred.reshape(-1), gt.reshape(-1), pb4.reshape(-1), vm4.reshape(-1))
    return out16[:, 0]
